# baseline (speedup 1.0000x reference)
import functools

import jax
import jax.numpy as jnp
from jax import lax
from jax.experimental import pallas as pl
from jax.experimental.pallas import tpu as pltpu
from jax.experimental.pallas import tpu_sc as plsc

_BB = 256


def _sc_gather(table, idx4, n_planes, b, h):
    info = plsc.get_sparse_core_info()
    nw = info.num_cores * info.num_subcores
    bw = b // nw
    cw = 2 * bw
    mesh = plsc.VectorSubcoreMesh(core_axis_name="c", subcore_axis_name="s")

    @functools.partial(
        pl.kernel,
        mesh=mesh,
        out_type=jax.ShapeDtypeStruct((n_planes, 2 * b, h), jnp.float32),
        compiler_params=pltpu.CompilerParams(use_tc_tiling_on_sc=False),
        scratch_types=[
            pltpu.VMEM((2, 128), jnp.int32),
            pltpu.VMEM((2, 128), jnp.int32),
            pltpu.VMEM((cw, h), jnp.float32),
            pltpu.VMEM((cw, h), jnp.float32),
            pltpu.SemaphoreType.DMA,
            pltpu.SemaphoreType.DMA,
        ],
    )
    def k(table_hbm, idx_hbm, out_hbm, idx0, idx1, rows0, rows1, sem0, sem1):
        wid = lax.axis_index("s") * info.num_cores + lax.axis_index("c")
        idx_v = (idx0, idx1)
        rows_v = (rows0, rows1)
        sems = (sem0, sem1)

        def fire(j, u):
            pltpu.sync_copy(idx_hbm.at[j, pl.ds(2 * wid, 2), :], idx_v[u])
            for q in range(2):
                pltpu.async_copy(
                    table_hbm.at[idx_v[u].at[q]],
                    rows_v[u].at[pl.ds(128 * q, 128), :],
                    sems[u])

        def drain(j, u):
            for q in range(2):
                pltpu.make_async_copy(
                    table_hbm.at[idx_v[u].at[q]],
                    rows_v[u].at[pl.ds(128 * q, 128), :],
                    sems[u]).wait()
            pltpu.sync_copy(rows_v[u], out_hbm.at[j, pl.ds(cw * wid, cw), :])

        fire(0, 0)

        def body(p, carry):
            g = 2 * p
            fire(g + 1, 1)
            drain(g, 0)
            fire(g + 2, 0)
            drain(g + 1, 1)
            return carry

        lax.fori_loop(0, (n_planes - 1) // 2, body, 0)
        drain(n_planes - 1, 0)

    return k(table, idx4)


def _dense_body(ce_ref, hid_ref, sc_ref, wq_ref, w2_ref, p2_ref, e_ref,
                attw_ref, out_ref):
    kp = ce_ref.shape[0]
    bB = ce_ref.shape[1]
    h = hid_ref.shape[1]
    lw = 2 * h
    q = jnp.dot(hid_ref[...], wq_ref[...], preferred_element_type=jnp.float32)
    q2 = jnp.concatenate([q, q], axis=1)
    pieces = []
    for j in range(kp):
        key = jnp.dot(ce_ref[j], w2_ref[...], preferred_element_type=jnp.float32)
        pieces.append(jnp.tanh(q2 + key))
    t_all = jnp.concatenate(pieces, axis=1)
    S = jnp.dot(t_all, p2_ref[...], preferred_element_type=jnp.float32)
    m = jnp.max(S, axis=1, keepdims=True)
    e = jnp.exp(S - m)
    l = jnp.sum(e, axis=1, keepdims=True)
    attw = e / l * sc_ref[...]
    attw_ref[...] = attw
    w_all = jnp.dot(attw, e_ref[...], preferred_element_type=jnp.float32)
    acc = jnp.zeros((bB, lw), jnp.float32)
    for j in range(kp):
        acc = acc + ce_ref[j] * w_all[:, j * lw:(j + 1) * lw]
    out_ref[...] = acc[:, :h] + acc[:, h:]


def _dense_tc(ce, hid, scores, wqT, w2, p2, e_all):
    kp, b, lw = ce.shape
    h = hid.shape[1]
    k = scores.shape[1]
    kl = kp * lw
    nb = b // _BB
    return pl.pallas_call(
        _dense_body,
        grid=(nb,),
        in_specs=[
            pl.BlockSpec((kp, _BB, lw), lambda i: (0, i, 0)),
            pl.BlockSpec((_BB, h), lambda i: (i, 0)),
            pl.BlockSpec((_BB, k), lambda i: (i, 0)),
            pl.BlockSpec((h, h), lambda i: (0, 0)),
            pl.BlockSpec((lw, lw), lambda i: (0, 0)),
            pl.BlockSpec((kl, k), lambda i: (0, 0)),
            pl.BlockSpec((k, kl), lambda i: (0, 0)),
        ],
        out_specs=[
            pl.BlockSpec((_BB, k), lambda i: (i, 0)),
            pl.BlockSpec((_BB, h), lambda i: (i, 0)),
        ],
        out_shape=[
            jax.ShapeDtypeStruct((b, k), jnp.float32),
            jax.ShapeDtypeStruct((b, h), jnp.float32),
        ],
    )(ce, hid, scores, wqT, w2, p2, e_all)


def _make_selectors(W_a, k, h):
    lw = 2 * h
    kl = (k // 2) * lw
    i = jnp.arange(kl)
    kcol = 2 * (i // lw) + (i % lw) // h
    wvals = jnp.tile(W_a[0], kl // h)
    p2 = jnp.where(kcol[:, None] == jnp.arange(k)[None, :],
                   wvals[:, None], 0.0)
    kk = jnp.arange(k)[:, None]
    cc = jnp.arange(kl)[None, :]
    e_all = ((cc // lw == kk // 2) & ((cc % lw) // h == kk % 2)
             ).astype(jnp.float32)
    return p2, e_all


def kernel(embedded, hidden, comp_ingr_id, scores, emb_table, W_q, W_k, W_a):
    b, k = comp_ingr_id.shape
    h = emb_table.shape[1]
    kp = k // 2
    idx4 = jnp.transpose(
        comp_ingr_id.astype(jnp.int32).reshape(b, kp, 2), (1, 0, 2)
    ).reshape(kp, (2 * b) // 128, 128)
    gathered = _sc_gather(emb_table, idx4, kp, b, h)
    ce = gathered.reshape(kp, b, 2 * h)
    wqT = W_q.T
    wkT = W_k.T
    w2 = jnp.zeros((2 * h, 2 * h), jnp.float32).at[:h, :h].set(wkT).at[h:, h:].set(wkT)
    p2, e_all = _make_selectors(W_a, k, h)
    attn_scores, attn = _dense_tc(ce, hidden[0], scores, wqT, w2, p2, e_all)
    output = jnp.concatenate([embedded[0], attn], axis=1)[None]
    return (output, attn_scores, comp_ingr_id)
